# Initial kernel scaffold; baseline (speedup 1.0000x reference)
import jax
import jax.numpy as jnp
from jax.experimental import pallas as pl


def kernel(x, edge_index, batch, ecfp, params):
    raise NotImplementedError("write your pallas kernel here")


# speedup vs baseline: 1.0000x; 1.0000x over previous
import jax
import jax.numpy as jnp
from jax.experimental import pallas as pl

N_GRAPHS = 256
HEADS = 4
HID = 256
CH = HID // HEADS


def _layer_norm(x, g, b, eps=1e-5):
    mu = jnp.mean(x, axis=-1, keepdims=True)
    var = jnp.mean((x - mu) ** 2, axis=-1, keepdims=True)
    return (x - mu) / jnp.sqrt(var + eps) * g + b


def _gat_conv(x, src, dst, p):
    N = x.shape[0]
    h = (x @ p["W"]).reshape(N, HEADS, CH)
    a_src = jnp.sum(h * p["att_src"][None, :, :], axis=-1)
    a_dst = jnp.sum(h * p["att_dst"][None, :, :], axis=-1)
    alpha = jax.nn.leaky_relu(a_src[src] + a_dst[dst], 0.2)
    amax = jax.ops.segment_max(alpha, dst, num_segments=N)
    amax = jnp.where(jnp.isneginf(amax), 0.0, amax)
    ex = jnp.exp(alpha - amax[dst])
    denom = jax.ops.segment_sum(ex, dst, num_segments=N)
    coef = ex / (denom[dst] + 1e-16)
    out = jax.ops.segment_sum(h[src] * coef[:, :, None], dst, num_segments=N)
    return out.reshape(N, HEADS * CH) + p["b"]


def _tail_kernel(g_ref, e_ref, fw_ref, fb_ref, fg_ref, fbeta_ref,
                 rw1_ref, rb1_ref, rw2_ref, rb2_ref, out_ref):
    g = g_ref[...]
    e = e_ref[...]
    emb = (g @ fw_ref[0:HID, :] + e @ fw_ref[HID:2 * HID, :]) + fb_ref[...]
    mu = jnp.mean(emb, axis=-1, keepdims=True)
    var = jnp.mean((emb - mu) ** 2, axis=-1, keepdims=True)
    emb = (emb - mu) / jnp.sqrt(var + 1e-5) * fg_ref[...] + fbeta_ref[...]
    emb = 0.5 * emb * (1.0 + jax.lax.erf(emb * 0.7071067811865476))
    h = jnp.maximum(emb @ rw1_ref[...] + rb1_ref[...], 0.0)
    out_ref[...] = h @ rw2_ref[...] + rb2_ref[...]


def kernel(x, edge_index, batch, ecfp, params):
    p = params
    N = x.shape[0]
    loop = jnp.arange(N, dtype=edge_index.dtype)
    src = jnp.concatenate([edge_index[0], loop])
    dst = jnp.concatenate([edge_index[1], loop])
    gx = jax.nn.elu(_gat_conv(x, src, dst, p["gat1"]))
    gx = jax.nn.elu(_gat_conv(gx, src, dst, p["gat2"]))
    gx = jax.nn.elu(_gat_conv(gx, src, dst, p["gat3"]))
    ssum = jax.ops.segment_sum(gx, batch, num_segments=N_GRAPHS)
    cnt = jax.ops.segment_sum(jnp.ones((N, 1), jnp.float32), batch, num_segments=N_GRAPHS)
    g_mean = ssum / jnp.maximum(cnt, 1.0)
    g_max = jax.ops.segment_max(gx, batch, num_segments=N_GRAPHS)
    g_max = jnp.where(jnp.isneginf(g_max), 0.0, g_max)
    g = (g_mean + g_max) / 2.0
    e = ecfp @ p["e_w1"] + p["e_b1"]
    e = _layer_norm(e, p["e_g1"], p["e_beta1"])
    e = jax.nn.gelu(e, approximate=False)
    e = e @ p["e_w2"] + p["e_b2"]
    e = _layer_norm(e, p["e_g2"], p["e_beta2"])
    e = jax.nn.gelu(e, approximate=False)
    out = pl.pallas_call(
        _tail_kernel,
        out_shape=jax.ShapeDtypeStruct((N_GRAPHS, 1), jnp.float32),
    )(g, e, p["f_w"], p["f_b"], p["f_g"], p["f_beta"],
      p["r_w1"], p["r_b1"], p["r_w2"], p["r_b2"])
    return out[:, 0]


# speedup vs baseline: 4.6273x; 4.6273x over previous
import jax
import jax.numpy as jnp
from jax.experimental import pallas as pl

N_GRAPHS = 256
HEADS = 4
HID = 256
CH = HID // HEADS
BN = 1024


def _att_matrix(att):
    a = jnp.zeros((HID, HEADS), jnp.float32)
    for hd in range(HEADS):
        a = a.at[hd * CH:(hd + 1) * CH, hd].set(att[hd])
    return a


def _node_first(x_ref, w_ref, asrc_ref, adst_ref, h_ref, s_ref, d_ref):
    h = x_ref[...] @ w_ref[...]
    h_ref[...] = h
    s_ref[...] = h @ asrc_ref[...]
    d_ref[...] = h @ adst_ref[...]


def _node_mid(z_ref, b_ref, w_ref, asrc_ref, adst_ref, h_ref, s_ref, d_ref):
    z = z_ref[...] + b_ref[...]
    act = jnp.where(z > 0, z, jnp.exp(jnp.minimum(z, 0.0)) - 1.0)
    h = act @ w_ref[...]
    h_ref[...] = h
    s_ref[...] = h @ asrc_ref[...]
    d_ref[...] = h @ adst_ref[...]


def _node_call(kfn, z, extras, in_dim):
    n = z.shape[0]
    grid = (pl.cdiv(n, BN),)
    in_specs = [pl.BlockSpec((BN, in_dim), lambda i: (i, 0))]
    in_specs += [pl.BlockSpec(e.shape, lambda i: tuple(0 for _ in e.shape))
                 for e in extras]
    return pl.pallas_call(
        kfn,
        grid=grid,
        in_specs=in_specs,
        out_specs=[pl.BlockSpec((BN, HID), lambda i: (i, 0)),
                   pl.BlockSpec((BN, HEADS), lambda i: (i, 0)),
                   pl.BlockSpec((BN, HEADS), lambda i: (i, 0))],
        out_shape=[jax.ShapeDtypeStruct((n, HID), jnp.float32),
                   jax.ShapeDtypeStruct((n, HEADS), jnp.float32),
                   jax.ShapeDtypeStruct((n, HEADS), jnp.float32)],
    )(z, *extras)


def _edge_softmax_agg(h, a_src, a_dst, src, dst, n):
    alpha = jax.nn.leaky_relu(a_src[src] + a_dst[dst], 0.2)
    amax = jax.ops.segment_max(alpha, dst, num_segments=n,
                               indices_are_sorted=True)
    amax = jnp.where(jnp.isneginf(amax), 0.0, amax)
    ex = jnp.exp(alpha - amax[dst])
    denom = jax.ops.segment_sum(ex, dst, num_segments=n,
                                indices_are_sorted=True)
    coef = ex / (denom[dst] + 1e-16)
    msg = (h[src].reshape(-1, HEADS, CH) * coef[:, :, None]).reshape(-1, HID)
    return jax.ops.segment_sum(msg, dst, num_segments=n,
                               indices_are_sorted=True)


def _tail(g_ref, ecfp_ref, ew1_ref, eb1_ref, eg1_ref, ebeta1_ref,
          ew2_ref, eb2_ref, eg2_ref, ebeta2_ref,
          fw_ref, fb_ref, fg_ref, fbeta_ref,
          rw1_ref, rb1_ref, rw2_ref, rb2_ref, out_ref):
    def ln(v, gam, bet):
        mu = jnp.mean(v, axis=-1, keepdims=True)
        var = jnp.mean((v - mu) ** 2, axis=-1, keepdims=True)
        return (v - mu) / jnp.sqrt(var + 1e-5) * gam + bet

    def gelu(v):
        return 0.5 * v * (1.0 + jax.lax.erf(v * 0.7071067811865476))

    e = ecfp_ref[...] @ ew1_ref[...] + eb1_ref[...]
    e = gelu(ln(e, eg1_ref[...], ebeta1_ref[...]))
    e = e @ ew2_ref[...] + eb2_ref[...]
    e = gelu(ln(e, eg2_ref[...], ebeta2_ref[...]))
    emb = g_ref[...] @ fw_ref[0:HID, :] + e @ fw_ref[HID:2 * HID, :] + fb_ref[...]
    emb = gelu(ln(emb, fg_ref[...], fbeta_ref[...]))
    h = jnp.maximum(emb @ rw1_ref[...] + rb1_ref[...], 0.0)
    out_ref[...] = h @ rw2_ref[...] + rb2_ref[...]


def kernel(x, edge_index, batch, ecfp, params):
    p = params
    n = x.shape[0]
    loop = jnp.arange(n, dtype=edge_index.dtype)
    src = jnp.concatenate([edge_index[0], loop])
    dst = jnp.concatenate([edge_index[1], loop])
    order = jnp.argsort(dst)
    src = src[order]
    dst = dst[order]

    a1s = _att_matrix(p["gat1"]["att_src"])
    a1d = _att_matrix(p["gat1"]["att_dst"])
    a2s = _att_matrix(p["gat2"]["att_src"])
    a2d = _att_matrix(p["gat2"]["att_dst"])
    a3s = _att_matrix(p["gat3"]["att_src"])
    a3d = _att_matrix(p["gat3"]["att_dst"])

    h1, s1, d1 = _node_call(_node_first, x, [p["gat1"]["W"], a1s, a1d], 9)
    o1 = _edge_softmax_agg(h1, s1, d1, src, dst, n)

    h2, s2, d2 = _node_call(
        _node_mid, o1,
        [p["gat1"]["b"][None, :], p["gat2"]["W"], a2s, a2d], HID)
    o2 = _edge_softmax_agg(h2, s2, d2, src, dst, n)

    h3, s3, d3 = _node_call(
        _node_mid, o2,
        [p["gat2"]["b"][None, :], p["gat3"]["W"], a3s, a3d], HID)
    o3 = _edge_softmax_agg(h3, s3, d3, src, dst, n)

    gx = jax.nn.elu(o3 + p["gat3"]["b"])

    ssum = jax.ops.segment_sum(gx, batch, num_segments=N_GRAPHS,
                               indices_are_sorted=True)
    cnt = jax.ops.segment_sum(jnp.ones((n, 1), jnp.float32), batch,
                              num_segments=N_GRAPHS, indices_are_sorted=True)
    g_mean = ssum / jnp.maximum(cnt, 1.0)
    g_max = jax.ops.segment_max(gx, batch, num_segments=N_GRAPHS,
                                indices_are_sorted=True)
    g_max = jnp.where(jnp.isneginf(g_max), 0.0, g_max)
    g = (g_mean + g_max) / 2.0

    out = pl.pallas_call(
        _tail,
        out_shape=jax.ShapeDtypeStruct((N_GRAPHS, 1), jnp.float32),
    )(g, ecfp, p["e_w1"], p["e_b1"], p["e_g1"], p["e_beta1"],
      p["e_w2"], p["e_b2"], p["e_g2"], p["e_beta2"],
      p["f_w"], p["f_b"], p["f_g"], p["f_beta"],
      p["r_w1"], p["r_b1"], p["r_w2"], p["r_b2"])
    return out[:, 0]
